# baseline (speedup 1.0000x reference)
import functools

import jax
import jax.numpy as jnp
from jax import lax
from jax.experimental import pallas as pl
from jax.experimental.pallas import tpu as pltpu
from jax.experimental.pallas import tpu_sc as plsc

_B = 16384
_DIM = 64
_GAMMA = 12.0
_NC = 2
_NS = 16
_NW = _NC * _NS
_BW = _B // _NW
_NCHUNK = 4
_CH = _BW // _NCHUNK
_RPB = 16
_NG = _BW // _RPB


def _score_body(heads_hbm, rel_hbm, tails_hbm, entT_hbm, relT_hbm, out_hbm,
                hidx, ridx, tidx, hbuf, rbuf, tbuf, outv, sem):
    wid = lax.axis_index("s") * _NC + lax.axis_index("c")
    base = wid * _BW

    for c in range(_NCHUNK):
        off = base + c * _CH
        pltpu.sync_copy(heads_hbm.at[pl.ds(off, _CH)], hidx.at[c])
        pltpu.sync_copy(rel_hbm.at[pl.ds(off, _CH)], ridx.at[c])
        pltpu.sync_copy(tails_hbm.at[pl.ds(off, _CH)], tidx.at[c])

    copies = []
    for d in range(_DIM):
        for c in range(_NCHUNK):
            dst = pl.ds(c * _CH, _CH)
            copies.append(pltpu.async_copy(
                entT_hbm.at[d].at[hidx.at[c]], hbuf.at[d, dst], sem))
            copies.append(pltpu.async_copy(
                relT_hbm.at[d].at[ridx.at[c]], rbuf.at[d, dst], sem))
            copies.append(pltpu.async_copy(
                entT_hbm.at[d].at[tidx.at[c]], tbuf.at[d, dst], sem))
    for cp in copies:
        cp.wait()

    def grp(g, carry):
        sl = pl.ds(g * _RPB, _RPB)
        acc = jnp.zeros((16,), jnp.float32)
        for d in range(_DIM):
            acc = acc + jnp.abs(hbuf[d, sl] + rbuf[d, sl] - tbuf[d, sl])
        outv[sl] = _GAMMA - acc
        return carry

    lax.fori_loop(0, _NG, grp, 0)
    pltpu.sync_copy(outv, out_hbm.at[pl.ds(base, _BW)])


@functools.partial(
    pl.kernel,
    out_type=jax.ShapeDtypeStruct((_B,), jnp.float32),
    mesh=plsc.VectorSubcoreMesh(core_axis_name="c", subcore_axis_name="s"),
    compiler_params=pltpu.CompilerParams(use_tc_tiling_on_sc=False),
    scratch_types=[
        pltpu.VMEM((_NCHUNK, _CH), jnp.int32),
        pltpu.VMEM((_NCHUNK, _CH), jnp.int32),
        pltpu.VMEM((_NCHUNK, _CH), jnp.int32),
        pltpu.VMEM((_DIM, _BW), jnp.float32),
        pltpu.VMEM((_DIM, _BW), jnp.float32),
        pltpu.VMEM((_DIM, _BW), jnp.float32),
        pltpu.VMEM((_BW,), jnp.float32),
        pltpu.SemaphoreType.DMA,
    ],
)
def _score(*refs):
    _score_body(*refs)


def kernel(heads, relations, tails, entity_embedding, relation_embedding):
    return _score(heads.astype(jnp.int32), relations.astype(jnp.int32),
                  tails.astype(jnp.int32), entity_embedding.T,
                  relation_embedding.T)


# speedup vs baseline: 4.9467x; 4.9467x over previous
import functools

import jax
import jax.numpy as jnp
from jax import lax
from jax.experimental import pallas as pl
from jax.experimental.pallas import tpu as pltpu
from jax.experimental.pallas import tpu_sc as plsc

_B = 16384
_DIM = 64
_GAMMA = 12.0
_NC = 2
_NS = 16
_NW = _NC * _NS
_BW = _B // _NW
_NCHUNK = 4
_CH = _BW // _NCHUNK
_HALF = 499968
_NLO = _HALF
_NHI = 1000000 - _HALF
_PASS = 256
_NPASS = _BW // _PASS
_RPB = 16
_NGP = _PASS // _RPB


def _lane_shuffle(x, idx):
    dnums = lax.GatherDimensionNumbers(
        offset_dims=(), collapsed_slice_dims=(0,), start_index_map=(0,))
    return lax.gather(x, idx[:, None], dnums, (1,),
                      mode=lax.GatherScatterMode.PROMISE_IN_BOUNDS)


def _score_body(heads_hbm, rel_hbm, tails_hbm, lo_hbm, hi_hbm, reltab_hbm,
                out_hbm, hraw, rraw, traw, hlo, hhi, tlo, thi,
                hloR, hhiR, tloR, thiR, rR, outv, sem):
    wid = lax.axis_index("s") * _NC + lax.axis_index("c")
    base = wid * _BW

    for c in range(_NCHUNK):
        off = base + c * _CH
        pltpu.sync_copy(heads_hbm.at[pl.ds(off, _CH)], hraw.at[c])
        pltpu.sync_copy(rel_hbm.at[pl.ds(off, _CH)], rraw.at[c])
        pltpu.sync_copy(tails_hbm.at[pl.ds(off, _CH)], traw.at[c])

    for c in range(_NCHUNK):
        for s8 in range(_CH // 16):
            sl = pl.ds(s8 * 16, 16)
            hv = hraw[c, sl]
            hlo[c, sl] = jnp.minimum(hv, _NLO - 1)
            hhi[c, sl] = jnp.maximum(hv - _NLO, 0)
            tv = traw[c, sl]
            tlo[c, sl] = jnp.minimum(tv, _NLO - 1)
            thi[c, sl] = jnp.maximum(tv - _NLO, 0)

    lane = lax.iota(jnp.int32, 16)

    for p in range(_NPASS):
        copies = []
        for c2 in range(_PASS // _CH):
            c = p * (_PASS // _CH) + c2
            dst = pl.ds(c2 * _CH, _CH)
            copies.append(pltpu.async_copy(lo_hbm.at[hlo.at[c]], hloR.at[dst], sem))
            copies.append(pltpu.async_copy(hi_hbm.at[hhi.at[c]], hhiR.at[dst], sem))
            copies.append(pltpu.async_copy(lo_hbm.at[tlo.at[c]], tloR.at[dst], sem))
            copies.append(pltpu.async_copy(hi_hbm.at[thi.at[c]], thiR.at[dst], sem))
            copies.append(pltpu.async_copy(reltab_hbm.at[rraw.at[c]], rR.at[dst], sem))
        for cp in copies:
            cp.wait()

        def grp(g, carry):
            c = p * (_PASS // _CH) + g // 8
            sub = (g % 8) * 16
            selh = jnp.clip(_NLO - hraw[c, pl.ds(sub, 16)], 0, 1).astype(jnp.float32)
            selt = jnp.clip(_NLO - traw[c, pl.ds(sub, 16)], 0, 1).astype(jnp.float32)
            z = jnp.zeros((16,), jnp.float32)
            o_ll, o_lh, o_hl, o_hh = z, z, z, z
            for ri in range(_RPB):
                row = g * _RPB + ri
                s_ll, s_lh, s_hl, s_hh = z, z, z, z
                for q in range(_DIM // 16):
                    sl = pl.ds(q * 16, 16)
                    a = hloR[row, sl] + rR[row, sl]
                    b = hhiR[row, sl] + rR[row, sl]
                    tl = tloR[row, sl]
                    th = thiR[row, sl]
                    s_ll = s_ll + jnp.abs(a - tl)
                    s_lh = s_lh + jnp.abs(a - th)
                    s_hl = s_hl + jnp.abs(b - tl)
                    s_hh = s_hh + jnp.abs(b - th)
                for sh in (8, 4, 2, 1):
                    s_ll = s_ll + _lane_shuffle(s_ll, lane ^ sh)
                    s_lh = s_lh + _lane_shuffle(s_lh, lane ^ sh)
                    s_hl = s_hl + _lane_shuffle(s_hl, lane ^ sh)
                    s_hh = s_hh + _lane_shuffle(s_hh, lane ^ sh)
                m = lane == ri
                o_ll = jnp.where(m, s_ll, o_ll)
                o_lh = jnp.where(m, s_lh, o_lh)
                o_hl = jnp.where(m, s_hl, o_hl)
                o_hh = jnp.where(m, s_hh, o_hh)
            blend_thi = o_hh + (o_lh - o_hh) * selh
            blend_tlo = o_hl + (o_ll - o_hl) * selh
            out16 = blend_thi + (blend_tlo - blend_thi) * selt
            outv[pl.ds(p * _PASS + g * _RPB, _RPB)] = _GAMMA - out16
            return carry

        lax.fori_loop(0, _NGP, grp, 0)

    pltpu.sync_copy(outv, out_hbm.at[pl.ds(base, _BW)])


@functools.partial(
    pl.kernel,
    out_type=jax.ShapeDtypeStruct((_B,), jnp.float32),
    mesh=plsc.VectorSubcoreMesh(core_axis_name="c", subcore_axis_name="s"),
    compiler_params=pltpu.CompilerParams(use_tc_tiling_on_sc=False),
    scratch_types=[
        pltpu.VMEM((_NCHUNK, _CH), jnp.int32),
        pltpu.VMEM((_NCHUNK, _CH), jnp.int32),
        pltpu.VMEM((_NCHUNK, _CH), jnp.int32),
        pltpu.VMEM((_NCHUNK, _CH), jnp.int32),
        pltpu.VMEM((_NCHUNK, _CH), jnp.int32),
        pltpu.VMEM((_NCHUNK, _CH), jnp.int32),
        pltpu.VMEM((_NCHUNK, _CH), jnp.int32),
        pltpu.VMEM((_PASS, _DIM), jnp.float32),
        pltpu.VMEM((_PASS, _DIM), jnp.float32),
        pltpu.VMEM((_PASS, _DIM), jnp.float32),
        pltpu.VMEM((_PASS, _DIM), jnp.float32),
        pltpu.VMEM((_PASS, _DIM), jnp.float32),
        pltpu.VMEM((_BW,), jnp.float32),
        pltpu.SemaphoreType.DMA,
    ],
)
def _score(*refs):
    _score_body(*refs)


def kernel(heads, relations, tails, entity_embedding, relation_embedding):
    return _score(heads.astype(jnp.int32), relations.astype(jnp.int32),
                  tails.astype(jnp.int32), entity_embedding[:_HALF],
                  entity_embedding[_HALF:], relation_embedding)


# speedup vs baseline: 8.1012x; 1.6377x over previous
import functools

import jax
import jax.numpy as jnp
from jax import lax
from jax.experimental import pallas as pl
from jax.experimental.pallas import tpu as pltpu
from jax.experimental.pallas import tpu_sc as plsc

_B = 16384
_DIM = 64
_GAMMA = 12.0
_NC = 2
_NS = 16
_NW = _NC * _NS
_BW = _B // _NW
_NCHUNK = 4
_CH = _BW // _NCHUNK
_PASS = 256
_CPP = _PASS // _CH
_NPASS = _BW // _PASS
_RPB = 16
_NGP = _PASS // _RPB


def _lane_shuffle(x, idx):
    dnums = lax.GatherDimensionNumbers(
        offset_dims=(), collapsed_slice_dims=(0,), start_index_map=(0,))
    return lax.gather(x, idx[:, None], dnums, (1,),
                      mode=lax.GatherScatterMode.PROMISE_IN_BOUNDS)


def _score_body(heads_hbm, rel_hbm, tails_hbm, ent2_hbm, rel2_hbm, out_hbm,
                hraw, rraw, traw, hrow, trow, h2, t2, r2, outv, sem):
    wid = lax.axis_index("s") * _NC + lax.axis_index("c")
    base = wid * _BW
    lane = lax.iota(jnp.int32, 16)

    for c in range(_NCHUNK):
        off = base + c * _CH
        pltpu.sync_copy(heads_hbm.at[pl.ds(off, _CH)], hraw.at[c])
        pltpu.sync_copy(rel_hbm.at[pl.ds(off, _CH)], rraw.at[c])
        pltpu.sync_copy(tails_hbm.at[pl.ds(off, _CH)], traw.at[c])

    for c in range(_NCHUNK):
        for s8 in range(_CH // 16):
            sl = pl.ds(s8 * 16, 16)
            hrow[c, sl] = lax.shift_right_logical(hraw[c, sl], 1)
            trow[c, sl] = lax.shift_right_logical(traw[c, sl], 1)

    for p in range(_NPASS):
        copies = []
        for c2 in range(_CPP):
            c = p * _CPP + c2
            dst = pl.ds(c2 * _CH, _CH)
            copies.append(pltpu.async_copy(ent2_hbm.at[hrow.at[c]], h2.at[dst], sem))
            copies.append(pltpu.async_copy(ent2_hbm.at[trow.at[c]], t2.at[dst], sem))
            copies.append(pltpu.async_copy(rel2_hbm.at[rraw.at[c]], r2.at[dst], sem))
        for cp in copies:
            cp.wait()

        def grp(g, carry):
            c = p * _CPP + g // 8
            sub = (g % 8) * 16
            wh = (1 - (hraw[c, pl.ds(sub, 16)] & 1)).astype(jnp.float32)
            wt = (1 - (traw[c, pl.ds(sub, 16)] & 1)).astype(jnp.float32)
            z = jnp.zeros((16,), jnp.float32)
            o00, o01, o10, o11 = z, z, z, z
            for ri in range(_RPB):
                row = g * _RPB + ri
                s00, s01, s10, s11 = z, z, z, z
                for q in range(_DIM // 16):
                    sle = pl.ds(q * 16, 16)
                    slo = pl.ds(_DIM + q * 16, 16)
                    rr = r2[row, sle]
                    a = h2[row, sle] + rr
                    b = h2[row, slo] + rr
                    te = t2[row, sle]
                    to = t2[row, slo]
                    s00 = s00 + jnp.abs(a - te)
                    s01 = s01 + jnp.abs(a - to)
                    s10 = s10 + jnp.abs(b - te)
                    s11 = s11 + jnp.abs(b - to)
                for sh in (8, 4, 2, 1):
                    s00 = s00 + _lane_shuffle(s00, lane ^ sh)
                    s01 = s01 + _lane_shuffle(s01, lane ^ sh)
                    s10 = s10 + _lane_shuffle(s10, lane ^ sh)
                    s11 = s11 + _lane_shuffle(s11, lane ^ sh)
                m = lane == ri
                o00 = jnp.where(m, s00, o00)
                o01 = jnp.where(m, s01, o01)
                o10 = jnp.where(m, s10, o10)
                o11 = jnp.where(m, s11, o11)
            b0 = o10 + (o00 - o10) * wh
            b1 = o11 + (o01 - o11) * wh
            out16 = b1 + (b0 - b1) * wt
            outv[pl.ds(p * _PASS + g * _RPB, _RPB)] = _GAMMA - out16
            return carry

        lax.fori_loop(0, _NGP, grp, 0)

    pltpu.sync_copy(outv, out_hbm.at[pl.ds(base, _BW)])


@functools.partial(
    pl.kernel,
    out_type=jax.ShapeDtypeStruct((_B,), jnp.float32),
    mesh=plsc.VectorSubcoreMesh(core_axis_name="c", subcore_axis_name="s"),
    compiler_params=pltpu.CompilerParams(use_tc_tiling_on_sc=True),
    scratch_types=[
        pltpu.VMEM((_NCHUNK, _CH), jnp.int32),
        pltpu.VMEM((_NCHUNK, _CH), jnp.int32),
        pltpu.VMEM((_NCHUNK, _CH), jnp.int32),
        pltpu.VMEM((_NCHUNK, _CH), jnp.int32),
        pltpu.VMEM((_NCHUNK, _CH), jnp.int32),
        pltpu.VMEM((_PASS, 2 * _DIM), jnp.float32),
        pltpu.VMEM((_PASS, 2 * _DIM), jnp.float32),
        pltpu.VMEM((_PASS, 2 * _DIM), jnp.float32),
        pltpu.VMEM((_BW,), jnp.float32),
        pltpu.SemaphoreType.DMA,
    ],
)
def _score(*refs):
    _score_body(*refs)


def kernel(heads, relations, tails, entity_embedding, relation_embedding):
    ent2 = entity_embedding.reshape(500000, 128)
    rel2 = jnp.concatenate([relation_embedding, relation_embedding], axis=1)
    return _score(heads.astype(jnp.int32), relations.astype(jnp.int32),
                  tails.astype(jnp.int32), ent2, rel2)


# speedup vs baseline: 9.0940x; 1.1225x over previous
import functools

import jax
import jax.numpy as jnp
from jax import lax
from jax.experimental import pallas as pl
from jax.experimental.pallas import tpu as pltpu
from jax.experimental.pallas import tpu_sc as plsc

_B = 16384
_DIM = 64
_GAMMA = 12.0
_NC = 2
_NS = 16
_NW = _NC * _NS
_BW = _B // _NW
_NCHUNK = 4
_CH = _BW // _NCHUNK
_PASS = 256
_CPP = _PASS // _CH
_NPASS = _BW // _PASS
_RPB = 16
_NGP = _PASS // _RPB


def _lane_shuffle(x, idx):
    dnums = lax.GatherDimensionNumbers(
        offset_dims=(), collapsed_slice_dims=(0,), start_index_map=(0,))
    return lax.gather(x, idx[:, None], dnums, (1,),
                      mode=lax.GatherScatterMode.PROMISE_IN_BOUNDS)


def _score_body(heads_hbm, rel_hbm, tails_hbm, ent2_hbm, rel2_hbm, out_hbm,
                hraw, rraw, traw, h2, t2, r2, outv, sem):
    wid = lax.axis_index("s") * _NC + lax.axis_index("c")
    base = wid * _BW
    lane = lax.iota(jnp.int32, 16)

    for c in range(_NCHUNK):
        off = base + c * _CH
        pltpu.sync_copy(heads_hbm.at[pl.ds(off, _CH)], hraw.at[c])
        pltpu.sync_copy(rel_hbm.at[pl.ds(off, _CH)], rraw.at[c])
        pltpu.sync_copy(tails_hbm.at[pl.ds(off, _CH)], traw.at[c])

    for p in range(_NPASS):
        copies = []
        for c2 in range(_CPP):
            c = p * _CPP + c2
            dst = pl.ds(c2 * _CH, _CH)
            copies.append(pltpu.async_copy(ent2_hbm.at[hraw.at[c]], h2.at[dst], sem))
            copies.append(pltpu.async_copy(ent2_hbm.at[traw.at[c]], t2.at[dst], sem))
            copies.append(pltpu.async_copy(rel2_hbm.at[rraw.at[c]], r2.at[dst], sem))
        for cp in copies:
            cp.wait()

        def grp(g, carry):
            out16 = jnp.zeros((16,), jnp.float32)
            for ri in range(_RPB):
                row = g * _RPB + ri
                s = jnp.zeros((16,), jnp.float32)
                for q in range(_DIM // 16):
                    sl = pl.ds(q * 16, 16)
                    s = s + jnp.abs(h2[row, sl] + r2[row, sl] - t2[row, sl])
                for sh in (8, 4, 2, 1):
                    s = s + _lane_shuffle(s, lane ^ sh)
                out16 = jnp.where(lane == ri, s, out16)
            outv[pl.ds(p * _PASS + g * _RPB, _RPB)] = _GAMMA - out16
            return carry

        lax.fori_loop(0, _NGP, grp, 0)

    pltpu.sync_copy(outv, out_hbm.at[pl.ds(base, _BW)])


@functools.partial(
    pl.kernel,
    out_type=jax.ShapeDtypeStruct((_B,), jnp.float32),
    mesh=plsc.VectorSubcoreMesh(core_axis_name="c", subcore_axis_name="s"),
    compiler_params=pltpu.CompilerParams(use_tc_tiling_on_sc=True),
    scratch_types=[
        pltpu.VMEM((_NCHUNK, _CH), jnp.int32),
        pltpu.VMEM((_NCHUNK, _CH), jnp.int32),
        pltpu.VMEM((_NCHUNK, _CH), jnp.int32),
        pltpu.VMEM((_PASS, 2 * _DIM), jnp.float32),
        pltpu.VMEM((_PASS, 2 * _DIM), jnp.float32),
        pltpu.VMEM((_PASS, 2 * _DIM), jnp.float32),
        pltpu.VMEM((_BW,), jnp.float32),
        pltpu.SemaphoreType.DMA,
    ],
)
def _score(*refs):
    _score_body(*refs)


def kernel(heads, relations, tails, entity_embedding, relation_embedding):
    ent2 = jnp.pad(entity_embedding, ((0, 0), (0, _DIM)))
    rel2 = jnp.pad(relation_embedding, ((0, 0), (0, _DIM)))
    return _score(heads.astype(jnp.int32), relations.astype(jnp.int32),
                  tails.astype(jnp.int32), ent2, rel2)
